# baseline (speedup 1.0000x reference)
import jax
import jax.numpy as jnp
from jax import lax
from jax.experimental import pallas as pl
from jax.experimental.pallas import tpu as pltpu
from jax.experimental.pallas import tpu_sc as plsc

BATCH = 16384
EMBED_DIM = 32
LANES = 16
CHUNK = 128
RATING_SCALE = 5.5

_info = plsc.get_sparse_core_info()
_NC, _NS = _info.num_cores, _info.num_subcores
NW = _NC * _NS
BPW = BATCH // NW
NCHUNK = BPW // CHUNK
NGROUP = BPW // LANES


def _ncf_body(users_hbm, items_hbm, wu_hbm, wi_hbm, bu_hbm, bi_hbm,
              out_hbm,
              u_idx, i_idx, u_rows, i_rows, u_b, i_b, pbuf, out_v, sem):
    wid = lax.axis_index("s") * _NC + lax.axis_index("c")
    base = wid * BPW

    pltpu.sync_copy(users_hbm.at[pl.ds(wid * NCHUNK, NCHUNK)], u_idx)
    pltpu.sync_copy(items_hbm.at[pl.ds(wid * NCHUNK, NCHUNK)], i_idx)

    handles = []
    for j in range(NCHUNK):
        sl = pl.ds(j * CHUNK, CHUNK)
        handles.append(pltpu.async_copy(wu_hbm.at[u_idx.at[j]], u_rows.at[sl], sem))
        handles.append(pltpu.async_copy(wi_hbm.at[i_idx.at[j]], i_rows.at[sl], sem))
        handles.append(pltpu.async_copy(bu_hbm.at[u_idx.at[j]], u_b.at[sl], sem))
        handles.append(pltpu.async_copy(bi_hbm.at[i_idx.at[j]], i_b.at[sl], sem))
    for h in handles:
        h.wait()

    def rowfold(r, carry):
        u0 = u_rows[r, pl.ds(0, LANES)]
        u1 = u_rows[r, pl.ds(LANES, LANES)]
        i0 = i_rows[r, pl.ds(0, LANES)]
        i1 = i_rows[r, pl.ds(LANES, LANES)]
        pbuf[pl.ds(r * LANES, LANES)] = u0 * i0 + u1 * i1
        return carry

    lax.fori_loop(0, BPW, rowfold, 0)

    lane = lax.iota(jnp.int32, LANES)

    def group(g, carry):
        rows = jnp.full((LANES,), g * LANES, jnp.int32) + lane
        flat = rows * LANES
        acc = jnp.zeros((LANES,), jnp.float32)
        for d in range(LANES):
            acc = acc + plsc.load_gather(pbuf, [flat + d])
        ub = u_b[pl.ds(g * LANES, LANES)]
        ib = i_b[pl.ds(g * LANES, LANES)]
        r = acc + ub + ib
        out_v[pl.ds(g * LANES, LANES)] = RATING_SCALE / (1.0 + jnp.exp(-r))
        return carry

    lax.fori_loop(0, NGROUP, group, 0)

    pltpu.sync_copy(out_v, out_hbm.at[pl.ds(base, BPW)])


def kernel(users, items, W_user, W_item, B_user, B_item):
    u = users.reshape(BATCH // CHUNK, CHUNK).astype(jnp.int32)
    it = items.reshape(BATCH // CHUNK, CHUNK).astype(jnp.int32)
    mesh = plsc.VectorSubcoreMesh(core_axis_name="c", subcore_axis_name="s")
    f = pl.kernel(
        _ncf_body,
        out_type=jax.ShapeDtypeStruct((BATCH,), jnp.float32),
        mesh=mesh,
        compiler_params=pltpu.CompilerParams(
            needs_layout_passes=False, use_tc_tiling_on_sc=False),
        scratch_types=[
            pltpu.VMEM((NCHUNK, CHUNK), jnp.int32),
            pltpu.VMEM((NCHUNK, CHUNK), jnp.int32),
            pltpu.VMEM((BPW, EMBED_DIM), jnp.float32),
            pltpu.VMEM((BPW, EMBED_DIM), jnp.float32),
            pltpu.VMEM((BPW,), jnp.float32),
            pltpu.VMEM((BPW,), jnp.float32),
            pltpu.VMEM((BPW * LANES,), jnp.float32),
            pltpu.VMEM((BPW,), jnp.float32),
            pltpu.SemaphoreType.DMA,
        ],
    )
    return f(u, it, W_user, W_item, B_user.reshape(-1), B_item.reshape(-1))
